# baseline (speedup 1.0000x reference)
import functools

import jax
import jax.numpy as jnp
from jax import lax
from jax.experimental import pallas as pl
from jax.experimental.pallas import tpu as pltpu
from jax.experimental.pallas import tpu_sc as plsc

_NC = 2
_NS = 16
_NW = _NC * _NS
_L = 16


@functools.partial(jax.jit, static_argnums=(3, 4, 5))
def _gather_mul(idx, weight, words, R, H, C):
  rpw = R // _NW
  nch = rpw // C
  mesh = plsc.VectorSubcoreMesh(core_axis_name="c", subcore_axis_name="s")

  @functools.partial(
      pl.kernel,
      out_type=jax.ShapeDtypeStruct((R, H), jnp.float32),
      mesh=mesh,
      scratch_types=[
          pltpu.VMEM((C,), jnp.int32),
          pltpu.VMEM((C, H), jnp.float32),
          pltpu.VMEM((C,), jnp.int32),
          pltpu.SemaphoreType.DMA,
          pltpu.SemaphoreType.DMA,
      ],
      compiler_params=pltpu.CompilerParams(
          needs_layout_passes=False, use_tc_tiling_on_sc=False
      ),
  )
  def gk(idx_hbm, w_hbm, mw_hbm, out_hbm, idx_v, w_v, m_v, sem_w, sem_m):
    wid = lax.axis_index("s") * _NC + lax.axis_index("c")
    lanes = lax.iota(jnp.int32, _L)

    def chunk(c, carry):
      base = wid * rpw + c * C
      pltpu.sync_copy(idx_hbm.at[pl.ds(base, C)], idx_v)
      cp_w = pltpu.async_copy(w_hbm.at[idx_v], w_v, sem_w)
      cp_m = pltpu.async_copy(mw_hbm.at[idx_v], m_v, sem_m)
      cp_w.wait()
      cp_m.wait()

      def row(j, carry2):
        word = plsc.load_gather(m_v, [jnp.full((_L,), j, jnp.int32)])
        bits = (word >> lanes) & 1
        w_v[j] = w_v[j] * bits.astype(jnp.float32)
        return carry2

      lax.fori_loop(0, C, row, 0)
      pltpu.sync_copy(w_v, out_hbm.at[pl.ds(base, C)])
      return carry

    lax.fori_loop(0, nch, chunk, 0)

  return gk(idx, weight, words)


def kernel(x, weight, mask):
  B, F = x.shape
  V, H = weight.shape
  R = B * F
  idx = x.reshape(R).astype(jnp.int32)
  pow2 = jnp.asarray([float(1 << i) for i in range(H)], dtype=jnp.float32)
  words = jnp.dot(mask.astype(jnp.float32), pow2).astype(jnp.int32)
  out = _gather_mul(idx, weight, words, R, H, 1664)
  return out.reshape(B, F, H)


# speedup vs baseline: 1.4992x; 1.4992x over previous
import functools

import jax
import jax.numpy as jnp
from jax import lax
from jax.experimental import pallas as pl
from jax.experimental.pallas import tpu as pltpu
from jax.experimental.pallas import tpu_sc as plsc

_NC = 2
_NS = 16
_NW = _NC * _NS
_L = 16


@functools.partial(jax.jit, static_argnums=(3, 4, 5, 6))
def _gather_mul(idx, weight, words, B, F, H, C):
  R = B * F
  cpf = B // C
  nch = R // (C * _NW)
  mesh = plsc.VectorSubcoreMesh(core_axis_name="c", subcore_axis_name="s")

  @functools.partial(
      pl.kernel,
      out_type=jax.ShapeDtypeStruct((F * H * B,), jnp.float32),
      mesh=mesh,
      scratch_types=[
          pltpu.VMEM((C,), jnp.int32),
          pltpu.VMEM((C, H), jnp.float32),
          pltpu.VMEM((C,), jnp.int32),
          pltpu.VMEM((H * C,), jnp.float32),
          pltpu.SemaphoreType.DMA,
          pltpu.SemaphoreType.DMA,
          pltpu.SemaphoreType.DMA,
      ],
      compiler_params=pltpu.CompilerParams(
          needs_layout_passes=False, use_tc_tiling_on_sc=False
      ),
  )
  def gk(idx_hbm, w_hbm, mw_hbm, out_hbm, idx_v, w_v, m_v, col_v, sem_w,
         sem_m, sem_o):
    wid = lax.axis_index("s") * _NC + lax.axis_index("c")
    lanes = lax.iota(jnp.int32, _L)
    lane_base = lanes * C

    def chunk(c, carry):
      q = wid * nch + c
      r0 = q * C
      pltpu.sync_copy(idx_hbm.at[pl.ds(r0, C)], idx_v)
      cp_w = pltpu.async_copy(w_hbm.at[idx_v], w_v, sem_w)
      cp_m = pltpu.async_copy(mw_hbm.at[idx_v], m_v, sem_m)
      cp_w.wait()
      cp_m.wait()

      def row(j, carry2):
        word = plsc.load_gather(m_v, [jnp.full((_L,), j, jnp.int32)])
        bits = (word >> lanes) & 1
        val = w_v[j] * bits.astype(jnp.float32)
        plsc.store_scatter(col_v, [lane_base + j], val)
        return carry2

      lax.fori_loop(0, C, row, 0)

      f = q // cpf
      b0 = (q % cpf) * C
      obase = f * (H * B) + b0
      cps = [
          pltpu.async_copy(
              col_v.at[pl.ds(h * C, C)],
              out_hbm.at[pl.ds(obase + h * B, C)],
              sem_o,
          )
          for h in range(H)
      ]
      for cp in cps:
        cp.wait()
      return carry

    lax.fori_loop(0, nch, chunk, 0)

  return gk(idx, weight, words)


def kernel(x, weight, mask):
  B, F = x.shape
  V, H = weight.shape
  idx = jnp.swapaxes(x, 0, 1).reshape(B * F).astype(jnp.int32)
  pow2 = jnp.asarray([float(1 << i) for i in range(H)], dtype=jnp.float32)
  words = jnp.dot(mask.astype(jnp.float32), pow2).astype(jnp.int32)
  flat = _gather_mul(idx, weight, words, B, F, H, 1024)
  return flat.reshape(F, H, B).transpose(2, 0, 1)


# speedup vs baseline: 1.7712x; 1.1814x over previous
import functools

import jax
import jax.numpy as jnp
from jax import lax
from jax.experimental import pallas as pl
from jax.experimental.pallas import tpu as pltpu
from jax.experimental.pallas import tpu_sc as plsc

_NC = 2
_NS = 16
_NW = _NC * _NS
_L = 16


@functools.partial(jax.jit, static_argnums=(3, 4, 5, 6))
def _gather_mul(idx, weight, words, B, F, H, C):
  R = B * F
  cpf = B // C
  nch = R // (C * _NW)
  mesh = plsc.VectorSubcoreMesh(core_axis_name="c", subcore_axis_name="s")

  @functools.partial(
      pl.kernel,
      out_type=jax.ShapeDtypeStruct((F * H * B,), jnp.float32),
      mesh=mesh,
      scratch_types=[
          [pltpu.VMEM((C,), jnp.int32)] * 2,
          [pltpu.VMEM((C, H), jnp.float32)] * 2,
          [pltpu.VMEM((C,), jnp.int32)] * 2,
          [pltpu.VMEM((H * C,), jnp.float32)] * 2,
          [pltpu.SemaphoreType.DMA] * 2,
          [pltpu.SemaphoreType.DMA] * 2,
          [pltpu.SemaphoreType.DMA] * 2,
      ],
      compiler_params=pltpu.CompilerParams(
          needs_layout_passes=False, use_tc_tiling_on_sc=False
      ),
  )
  def gk(idx_hbm, w_hbm, mw_hbm, out_hbm, idx_v, w_v, m_v, col_v, sem_w,
         sem_m, sem_o):
    wid = lax.axis_index("s") * _NC + lax.axis_index("c")
    lanes = lax.iota(jnp.int32, _L)
    lane_base = lanes * C
    q0 = wid * nch

    def start_gathers(c):
      p = c % 2
      pltpu.sync_copy(idx_hbm.at[pl.ds((q0 + c) * C, C)], idx_v[p])
      cp_w = pltpu.async_copy(w_hbm.at[idx_v[p]], w_v[p], sem_w[p])
      cp_m = pltpu.async_copy(mw_hbm.at[idx_v[p]], m_v[p], sem_m[p])
      return cp_w, cp_m

    pending_gather = {0: start_gathers(0)}
    pending_out = {}

    for c in range(nch):
      p = c % 2
      if c + 1 < nch:
        pending_gather[c + 1] = start_gathers(c + 1)
      cp_w, cp_m = pending_gather.pop(c)
      cp_w.wait()
      cp_m.wait()
      if c - 2 in pending_out:
        for cp in pending_out.pop(c - 2):
          cp.wait()

      @plsc.parallel_loop(0, C, 1, unroll=4)
      def _row(j):
        word = plsc.load_gather(m_v[p], [jnp.full((_L,), j, jnp.int32)])
        bits = (word >> lanes) & 1
        val = w_v[p][j] * bits.astype(jnp.float32)
        plsc.store_scatter(col_v[p], [lane_base + j], val)

      q = q0 + c
      f = q // cpf
      b0 = (q % cpf) * C
      obase = f * (H * B) + b0
      pending_out[c] = [
          pltpu.async_copy(
              col_v[p].at[pl.ds(h * C, C)],
              out_hbm.at[pl.ds(obase + h * B, C)],
              sem_o[p],
          )
          for h in range(H)
      ]
    for cps in pending_out.values():
      for cp in cps:
        cp.wait()

  return gk(idx, weight, words)


def kernel(x, weight, mask):
  B, F = x.shape
  V, H = weight.shape
  idx = jnp.swapaxes(x, 0, 1).reshape(B * F).astype(jnp.int32)
  pow2 = jnp.asarray([float(1 << i) for i in range(H)], dtype=jnp.float32)
  words = jnp.dot(mask.astype(jnp.float32), pow2).astype(jnp.int32)
  flat = _gather_mul(idx, weight, words, B, F, H, 1024)
  return flat.reshape(F, H, B).transpose(2, 0, 1)
